# baseline (speedup 1.0000x reference)
import functools

import jax
import jax.numpy as jnp
from jax import lax
from jax.experimental import pallas as pl
from jax.experimental.pallas import tpu as pltpu
from jax.experimental.pallas import tpu_sc as plsc

_KT = 1024
_KBLK = 49


def _pack_body(a_ref, b_ref, o_ref):
    both = jnp.concatenate([a_ref[...], b_ref[...]], axis=0)
    eye = jnp.eye(both.shape[0], dtype=jnp.float32)
    o_ref[...] = lax.dot_general(
        both, eye, (((0,), (0,)), ((), ())),
        preferred_element_type=jnp.float32,
    )


@functools.lru_cache(maxsize=None)
def _make_sc_gather(K, D2, B):
    info = plsc.get_sparse_core_info()
    NC, NS, L = info.num_cores, info.num_subcores, info.num_lanes
    NW = NC * NS
    assert B % NW == 0 and (B // NW) % 8 == 0 and (B // NW) % L == 0
    b_per_w = B // NW
    mesh = plsc.VectorSubcoreMesh(core_axis_name="c", subcore_axis_name="s")

    @functools.partial(
        pl.kernel,
        out_type=jax.ShapeDtypeStruct((B, D2), jnp.float32),
        mesh=mesh,
        scratch_types=[
            pltpu.VMEM((b_per_w,), jnp.int32),
            pltpu.VMEM((b_per_w,), jnp.int32),
            pltpu.VMEM((b_per_w, D2), jnp.float32),
            pltpu.SemaphoreType.DMA,
        ],
    )
    def gather_kernel(idx_hbm, table_hbm, out_hbm, idx_v, idx2_v, rows_v,
                      sem):
        wid = lax.axis_index("s") * NC + lax.axis_index("c")
        base = wid * b_per_w
        pltpu.sync_copy(idx_hbm.at[pl.ds(base, b_per_w)], idx_v)
        for c in range(b_per_w // L):
            sl = pl.ds(c * L, L)
            v = idx_v[sl]
            idx2_v[sl] = v - jnp.where(v >= K, K, 0).astype(jnp.int32)
        pltpu.async_copy(table_hbm.at[idx2_v], rows_v, sem).wait()
        pltpu.sync_copy(rows_v, out_hbm.at[pl.ds(base, b_per_w)])

    return gather_kernel


def _proj_body(x2_ref, sel_ref, w_ref, b_ref, o_ref, xr_ref):
    D = w_ref.shape[0]

    @pl.when(pl.program_id(0) == 0)
    def _make_x():
        x2 = x2_ref[...]
        sel = sel_ref[...] > 0
        xv = jnp.where(sel, x2[:, D:], x2[:, :D])
        xr_ref[...] = jnp.maximum(xv, 0.0)

    xr = xr_ref[...]
    ones_row = jnp.ones((1, xr.shape[0]), jnp.float32)
    bias = lax.dot_general(
        b_ref[...], ones_row, (((0,), (0,)), ((), ())),
        preferred_element_type=jnp.float32,
    )
    o_ref[...] = lax.dot_general(
        w_ref[...], xr, (((0,), (1,)), ((), ())),
        preferred_element_type=jnp.float32,
    ) + bias


def kernel(text, emb_table, fc_w, fc_b):
    B = text.shape[0]
    V, D = fc_w.shape
    K = _KBLK * _KT
    assert K < V <= 2 * K

    text = text.astype(jnp.int32)
    emb_t = emb_table.T

    table2 = pl.pallas_call(
        _pack_body,
        grid=(_KBLK,),
        in_specs=[
            pl.BlockSpec((D, _KT), lambda j: (0, j)),
            pl.BlockSpec((D, _KT), lambda j: (0, j + _KBLK)),
        ],
        out_specs=pl.BlockSpec((_KT, 2 * D), lambda j: (j, 0)),
        out_shape=jax.ShapeDtypeStruct((K, 2 * D), jnp.float32),
        compiler_params=pltpu.CompilerParams(
            dimension_semantics=("arbitrary",),
        ),
    )(emb_t, emb_t)

    x2 = _make_sc_gather(K, 2 * D, B)(text, table2)
    sel = (text >= K).astype(jnp.int32).reshape(B, 1)

    VT = 2048
    out_t = pl.pallas_call(
        _proj_body,
        grid=(pl.cdiv(V, VT),),
        in_specs=[
            pl.BlockSpec((B, 2 * D), lambda j: (0, 0)),
            pl.BlockSpec((B, 1), lambda j: (0, 0)),
            pl.BlockSpec((D, VT), lambda j: (0, j)),
            pl.BlockSpec((1, VT), lambda j: (0, j)),
        ],
        out_specs=pl.BlockSpec((VT, B), lambda j: (j, 0)),
        out_shape=jax.ShapeDtypeStruct((V, B), jnp.float32),
        scratch_shapes=[pltpu.VMEM((B, D), jnp.float32)],
        compiler_params=pltpu.CompilerParams(
            dimension_semantics=("arbitrary",),
        ),
    )(x2, sel, fc_w.T, fc_b.reshape(1, V))
    return out_t.T


# speedup vs baseline: 1.0234x; 1.0234x over previous
import functools

import jax
import jax.numpy as jnp
from jax import lax
from jax.experimental import pallas as pl
from jax.experimental.pallas import tpu as pltpu
from jax.experimental.pallas import tpu_sc as plsc

_KT = 1024
_KBLK = 49


def _pack_body(a_ref, b_ref, o_ref):
    o_ref[...] = jnp.transpose(
        jnp.concatenate([a_ref[...], b_ref[...]], axis=0), (1, 0))


@functools.lru_cache(maxsize=None)
def _make_sc_gather(K, D2, B):
    info = plsc.get_sparse_core_info()
    NC, NS, L = info.num_cores, info.num_subcores, info.num_lanes
    NW = NC * NS
    assert B % NW == 0 and (B // NW) % 8 == 0 and (B // NW) % L == 0
    b_per_w = B // NW
    mesh = plsc.VectorSubcoreMesh(core_axis_name="c", subcore_axis_name="s")

    @functools.partial(
        pl.kernel,
        out_type=jax.ShapeDtypeStruct((B, D2), jnp.float32),
        mesh=mesh,
        scratch_types=[
            pltpu.VMEM((b_per_w,), jnp.int32),
            pltpu.VMEM((b_per_w,), jnp.int32),
            pltpu.VMEM((b_per_w, D2), jnp.float32),
            pltpu.SemaphoreType.DMA,
        ],
    )
    def gather_kernel(idx_hbm, table_hbm, out_hbm, idx_v, idx2_v, rows_v,
                      sem):
        wid = lax.axis_index("s") * NC + lax.axis_index("c")
        base = wid * b_per_w
        pltpu.sync_copy(idx_hbm.at[pl.ds(base, b_per_w)], idx_v)
        for c in range(b_per_w // L):
            sl = pl.ds(c * L, L)
            v = idx_v[sl]
            idx2_v[sl] = v - jnp.where(v >= K, K, 0).astype(jnp.int32)
        pltpu.async_copy(table_hbm.at[idx2_v], rows_v, sem).wait()
        pltpu.sync_copy(rows_v, out_hbm.at[pl.ds(base, b_per_w)])

    return gather_kernel


def _proj_body(x2_ref, sel_ref, w_ref, b_ref, o_ref, xr_ref):
    D = w_ref.shape[0]

    @pl.when(pl.program_id(0) == 0)
    def _make_x():
        x2 = x2_ref[...]
        sel = sel_ref[...] > 0
        xv = jnp.where(sel, x2[:, D:], x2[:, :D])
        xr_ref[...] = jnp.maximum(xv, 0.0)

    xr = xr_ref[...]
    ones_row = jnp.ones((1, xr.shape[0]), jnp.float32)
    bias = lax.dot_general(
        b_ref[...], ones_row, (((0,), (0,)), ((), ())),
        preferred_element_type=jnp.float32,
    )
    o_ref[...] = lax.dot_general(
        w_ref[...], xr, (((0,), (1,)), ((), ())),
        preferred_element_type=jnp.float32,
    ) + bias


def kernel(text, emb_table, fc_w, fc_b):
    B = text.shape[0]
    V, D = fc_w.shape
    K = _KBLK * _KT
    assert K < V <= 2 * K

    text = text.astype(jnp.int32)
    emb_t = emb_table.T

    table2 = pl.pallas_call(
        _pack_body,
        grid=(_KBLK,),
        in_specs=[
            pl.BlockSpec((D, _KT), lambda j: (0, j)),
            pl.BlockSpec((D, _KT), lambda j: (0, j + _KBLK)),
        ],
        out_specs=pl.BlockSpec((_KT, 2 * D), lambda j: (j, 0)),
        out_shape=jax.ShapeDtypeStruct((K, 2 * D), jnp.float32),
        compiler_params=pltpu.CompilerParams(
            dimension_semantics=("arbitrary",),
        ),
    )(emb_t, emb_t)

    x2 = _make_sc_gather(K, 2 * D, B)(text, table2)
    sel = (text >= K).astype(jnp.int32).reshape(B, 1)

    VT = 4096
    out_t = pl.pallas_call(
        _proj_body,
        grid=(pl.cdiv(V, VT),),
        in_specs=[
            pl.BlockSpec((B, 2 * D), lambda j: (0, 0)),
            pl.BlockSpec((B, 1), lambda j: (0, 0)),
            pl.BlockSpec((D, VT), lambda j: (0, j)),
            pl.BlockSpec((1, VT), lambda j: (0, j)),
        ],
        out_specs=pl.BlockSpec((VT, B), lambda j: (j, 0)),
        out_shape=jax.ShapeDtypeStruct((V, B), jnp.float32),
        scratch_shapes=[pltpu.VMEM((B, D), jnp.float32)],
        compiler_params=pltpu.CompilerParams(
            dimension_semantics=("arbitrary",),
        ),
    )(x2, sel, fc_w.T, fc_b.reshape(1, V))
    return out_t.T


# speedup vs baseline: 1.0828x; 1.0581x over previous
import functools

import jax
import jax.numpy as jnp
from jax import lax
from jax.experimental import pallas as pl
from jax.experimental.pallas import tpu as pltpu
from jax.experimental.pallas import tpu_sc as plsc

_KT = 2048
_KBLK = 25


def _pack_body(a_ref, b_ref, o_ref):
    o_ref[...] = jnp.transpose(
        jnp.concatenate([a_ref[...], b_ref[...]], axis=0), (1, 0))


@functools.lru_cache(maxsize=None)
def _make_sc_gather(K, D2, B):
    info = plsc.get_sparse_core_info()
    NC, NS, L = info.num_cores, info.num_subcores, info.num_lanes
    NW = NC * NS
    assert B % NW == 0 and (B // NW) % 8 == 0 and (B // NW) % L == 0
    b_per_w = B // NW
    mesh = plsc.VectorSubcoreMesh(core_axis_name="c", subcore_axis_name="s")

    @functools.partial(
        pl.kernel,
        out_type=jax.ShapeDtypeStruct((B, D2), jnp.float32),
        mesh=mesh,
        scratch_types=[
            pltpu.VMEM((b_per_w,), jnp.int32),
            pltpu.VMEM((b_per_w,), jnp.int32),
            pltpu.VMEM((b_per_w, D2), jnp.float32),
            pltpu.SemaphoreType.DMA,
        ],
    )
    def gather_kernel(idx_hbm, table_hbm, out_hbm, idx_v, idx2_v, rows_v,
                      sem):
        wid = lax.axis_index("s") * NC + lax.axis_index("c")
        base = wid * b_per_w
        pltpu.sync_copy(idx_hbm.at[pl.ds(base, b_per_w)], idx_v)
        for c in range(b_per_w // L):
            sl = pl.ds(c * L, L)
            v = idx_v[sl]
            idx2_v[sl] = v - jnp.where(v >= K, K, 0).astype(jnp.int32)
        pltpu.async_copy(table_hbm.at[idx2_v], rows_v, sem).wait()
        pltpu.sync_copy(rows_v, out_hbm.at[pl.ds(base, b_per_w)])

    return gather_kernel


def _proj_body(x2_ref, sel_ref, w_ref, b_ref, o_ref, xr_ref):
    D = w_ref.shape[0]

    @pl.when(pl.program_id(0) == 0)
    def _make_x():
        x2 = x2_ref[...]
        sel = sel_ref[...] > 0
        xv = jnp.where(sel, x2[:, D:], x2[:, :D])
        xr_ref[...] = jnp.maximum(xv, 0.0)

    xr = xr_ref[...]
    ones_row = jnp.ones((1, xr.shape[0]), jnp.float32)
    bias = lax.dot_general(
        b_ref[...], ones_row, (((0,), (0,)), ((), ())),
        preferred_element_type=jnp.float32,
    )
    o_ref[...] = lax.dot_general(
        w_ref[...], xr, (((0,), (1,)), ((), ())),
        preferred_element_type=jnp.float32,
    ) + bias


def kernel(text, emb_table, fc_w, fc_b):
    B = text.shape[0]
    V, D = fc_w.shape
    K = _KBLK * _KT
    assert K < V <= 2 * K

    text = text.astype(jnp.int32)
    emb_t = emb_table.T

    last_blk = (V - 1) // _KT
    table2 = pl.pallas_call(
        _pack_body,
        grid=(_KBLK,),
        in_specs=[
            pl.BlockSpec((D, _KT), lambda j: (0, j)),
            pl.BlockSpec((D, _KT),
                         lambda j: (0, jnp.minimum(j + _KBLK, last_blk))),
        ],
        out_specs=pl.BlockSpec((_KT, 2 * D), lambda j: (j, 0)),
        out_shape=jax.ShapeDtypeStruct((K, 2 * D), jnp.float32),
        compiler_params=pltpu.CompilerParams(
            dimension_semantics=("arbitrary",),
        ),
    )(emb_t, emb_t)

    x2 = _make_sc_gather(K, 2 * D, B)(text, table2)
    sel = (text >= K).astype(jnp.int32).reshape(B, 1)

    VT = 4096
    out_t = pl.pallas_call(
        _proj_body,
        grid=(pl.cdiv(V, VT),),
        in_specs=[
            pl.BlockSpec((B, 2 * D), lambda j: (0, 0)),
            pl.BlockSpec((B, 1), lambda j: (0, 0)),
            pl.BlockSpec((D, VT), lambda j: (0, j)),
            pl.BlockSpec((1, VT), lambda j: (0, j)),
        ],
        out_specs=pl.BlockSpec((VT, B), lambda j: (j, 0)),
        out_shape=jax.ShapeDtypeStruct((V, B), jnp.float32),
        scratch_shapes=[pltpu.VMEM((B, D), jnp.float32)],
        compiler_params=pltpu.CompilerParams(
            dimension_semantics=("arbitrary",),
        ),
    )(x2, sel, fc_w.T, fc_b.reshape(1, V))
    return out_t.T


# speedup vs baseline: 1.1243x; 1.0383x over previous
import functools

import jax
import jax.numpy as jnp
from jax import lax
from jax.experimental import pallas as pl
from jax.experimental.pallas import tpu as pltpu
from jax.experimental.pallas import tpu_sc as plsc

_KT = 4096
_KBLK = 13


def _pack_body(a_ref, b_ref, o_ref):
    o_ref[...] = jnp.transpose(
        jnp.concatenate([a_ref[...], b_ref[...]], axis=0), (1, 0))


@functools.lru_cache(maxsize=None)
def _make_sc_gather(K, D2, B):
    info = plsc.get_sparse_core_info()
    NC, NS, L = info.num_cores, info.num_subcores, info.num_lanes
    NW = NC * NS
    assert B % NW == 0 and (B // NW) % 8 == 0 and (B // NW) % L == 0
    b_per_w = B // NW
    mesh = plsc.VectorSubcoreMesh(core_axis_name="c", subcore_axis_name="s")

    @functools.partial(
        pl.kernel,
        out_type=jax.ShapeDtypeStruct((B, D2), jnp.float32),
        mesh=mesh,
        scratch_types=[
            pltpu.VMEM((b_per_w,), jnp.int32),
            pltpu.VMEM((b_per_w,), jnp.int32),
            pltpu.VMEM((b_per_w, D2), jnp.float32),
            pltpu.SemaphoreType.DMA,
        ],
    )
    def gather_kernel(idx_hbm, table_hbm, out_hbm, idx_v, idx2_v, rows_v,
                      sem):
        wid = lax.axis_index("s") * NC + lax.axis_index("c")
        base = wid * b_per_w
        pltpu.sync_copy(idx_hbm.at[pl.ds(base, b_per_w)], idx_v)
        for c in range(b_per_w // L):
            sl = pl.ds(c * L, L)
            v = idx_v[sl]
            idx2_v[sl] = v - jnp.where(v >= K, K, 0).astype(jnp.int32)
        pltpu.async_copy(table_hbm.at[idx2_v], rows_v, sem).wait()
        pltpu.sync_copy(rows_v, out_hbm.at[pl.ds(base, b_per_w)])

    return gather_kernel


def _proj_body(x2_ref, sel_ref, w_ref, b_ref, o_ref, xr_ref):
    D = w_ref.shape[0]

    @pl.when(pl.program_id(0) == 0)
    def _make_x():
        x2 = x2_ref[...]
        sel = sel_ref[...] > 0
        xv = jnp.where(sel, x2[:, D:], x2[:, :D])
        xr_ref[...] = jnp.maximum(xv, 0.0)

    xr = xr_ref[...]
    ones_row = jnp.ones((1, xr.shape[0]), jnp.float32)
    bias = lax.dot_general(
        b_ref[...], ones_row, (((0,), (0,)), ((), ())),
        preferred_element_type=jnp.float32,
    )
    o_ref[...] = lax.dot_general(
        w_ref[...], xr, (((0,), (1,)), ((), ())),
        preferred_element_type=jnp.float32,
    ) + bias


def kernel(text, emb_table, fc_w, fc_b):
    B = text.shape[0]
    V, D = fc_w.shape
    K = _KBLK * _KT
    assert K < V <= 2 * K

    text = text.astype(jnp.int32)
    emb_t = emb_table.T

    last_blk = (V - 1) // _KT
    table2 = pl.pallas_call(
        _pack_body,
        grid=(_KBLK,),
        in_specs=[
            pl.BlockSpec((D, _KT), lambda j: (0, j)),
            pl.BlockSpec((D, _KT),
                         lambda j: (0, jnp.minimum(j + _KBLK, last_blk))),
        ],
        out_specs=pl.BlockSpec((_KT, 2 * D), lambda j: (j, 0)),
        out_shape=jax.ShapeDtypeStruct((K, 2 * D), jnp.float32),
        compiler_params=pltpu.CompilerParams(
            dimension_semantics=("arbitrary",),
        ),
    )(emb_t, emb_t)

    x2 = _make_sc_gather(K, 2 * D, B)(text, table2)
    sel = (text >= K).astype(jnp.int32).reshape(B, 1)

    VT = 4096
    out_t = pl.pallas_call(
        _proj_body,
        grid=(pl.cdiv(V, VT),),
        in_specs=[
            pl.BlockSpec((B, 2 * D), lambda j: (0, 0)),
            pl.BlockSpec((B, 1), lambda j: (0, 0)),
            pl.BlockSpec((D, VT), lambda j: (0, j)),
            pl.BlockSpec((1, VT), lambda j: (0, j)),
        ],
        out_specs=pl.BlockSpec((VT, B), lambda j: (j, 0)),
        out_shape=jax.ShapeDtypeStruct((V, B), jnp.float32),
        scratch_shapes=[pltpu.VMEM((B, D), jnp.float32)],
        compiler_params=pltpu.CompilerParams(
            dimension_semantics=("arbitrary",),
        ),
    )(x2, sel, fc_w.T, fc_b.reshape(1, V))
    return out_t.T
